# baseline (speedup 1.0000x reference)
import functools

import jax
import jax.numpy as jnp
import numpy as np
from jax import lax
from jax.experimental import pallas as pl
from jax.experimental.pallas import tpu as pltpu
import jax.experimental.pallas.tpu_sc as plsc

_N = 10000
_E = 320000
_F = 128
_De = 16
_H = 128
_G = 100
_NCONV = 20
_NB = 4
_NE4 = _NB * _E
_N4 = _NB * _N

_TN = 2000
_TE = 2000
_GP = 104

_NW = 32
_EPW = _NE4 // _NW
_CEG = 200
_IBLK = 4000
_NBLK = _EPW // _IBLK
_NPRB = _IBLK // (2 * _CEG)
_EPS = _E // _NW
_CES = 80
_NCHS = _EPS // _CES
_NPRS = (_NCHS - 1) // 2
_NP = 10240
_NPS = _NP // 16

_f32 = jnp.float32
_u32 = jnp.uint32
_bf16 = jnp.bfloat16


def _silu(x):
    return x * jax.nn.sigmoid(x)


def _prep_body(h_ref, a_ref, b_ref, ps_ref, pd_ref):
    h = h_ref[...]
    ps_ref[...] = jnp.dot(h, a_ref[...], preferred_element_type=_f32)
    pd_ref[...] = jnp.dot(h, b_ref[...], preferred_element_type=_f32)


def _tc_prep(h, A, B):
    nt = _N // _TN
    bi = lambda b, i: (b * nt + i, 0)
    wb = lambda b, i: (b, 0)
    return pl.pallas_call(
        _prep_body,
        grid=(_NB, nt),
        in_specs=[
            pl.BlockSpec((_TN, _F), bi),
            pl.BlockSpec((_F, _F), wb),
            pl.BlockSpec((_F, _F), wb),
        ],
        out_specs=[
            pl.BlockSpec((_TN, _F), bi),
            pl.BlockSpec((_TN, _F), bi),
        ],
        out_shape=[jax.ShapeDtypeStruct((_N4, _F), _f32)] * 2,
    )(h, A, B)


def _pack_table(P, xp):
    pw = lax.bitcast_convert_type(
        P.astype(_bf16).reshape(_N4, _F // 2, 2), _u32)
    xw = lax.bitcast_convert_type(xp[:, 0:3], _u32)
    return jnp.concatenate(
        [pw, xw, jnp.zeros((_N4, 61), _u32)], axis=1)


_SC_MESH = plsc.VectorSubcoreMesh(core_axis_name="c", subcore_axis_name="s")


@functools.partial(
    pl.kernel,
    out_type=[
        jax.ShapeDtypeStruct((_NE4, _F), _u32),
        jax.ShapeDtypeStruct((_NE4, _F), _u32),
    ],
    mesh=_SC_MESH,
    scratch_types=[
        pltpu.VMEM((_IBLK,), jnp.int32),
        pltpu.VMEM((_IBLK,), jnp.int32),
        pltpu.VMEM((_CEG, _F), _u32),
        pltpu.VMEM((_CEG, _F), _u32),
        pltpu.VMEM((_CEG, _F), _u32),
        pltpu.VMEM((_CEG, _F), _u32),
        pltpu.SemaphoreType.DMA,
        pltpu.SemaphoreType.DMA,
        pltpu.SemaphoreType.DMA,
        pltpu.SemaphoreType.DMA,
    ],
)
def _sc_gather(ts_hbm, td_hbm, src_hbm, dst_hbm,
               gs_hbm, gd_hbm,
               isv, idv, bAs, bAd, bBs, bBd, sgA, sgB, swA, swB):
    cid = lax.axis_index("c")
    sid = lax.axis_index("s")
    wbase = pl.multiple_of((cid * 16 + sid) * _EPW, 8)

    def issue_g(bs, bd, k, sem):
        off = pl.multiple_of(k * _CEG, 8)
        pltpu.async_copy(ts_hbm.at[isv.at[pl.ds(off, _CEG)]], bs, sem)
        pltpu.async_copy(td_hbm.at[idv.at[pl.ds(off, _CEG)]], bd, sem)

    def wait_g(bs, bd, sem):
        pltpu.make_async_copy(ts_hbm.at[pl.ds(0, _CEG)], bs, sem).wait()
        pltpu.make_async_copy(td_hbm.at[pl.ds(0, _CEG)], bd, sem).wait()

    def issue_w(bs, bd, boff, k, sem):
        base = pl.multiple_of(wbase + boff + k * _CEG, 8)
        pltpu.async_copy(bs, gs_hbm.at[pl.ds(base, _CEG)], sem)
        pltpu.async_copy(bd, gd_hbm.at[pl.ds(base, _CEG)], sem)

    def wait_w(bs, bd, sem):
        pltpu.make_async_copy(bs, gs_hbm.at[pl.ds(wbase, _CEG)], sem).wait()
        pltpu.make_async_copy(bd, gd_hbm.at[pl.ds(wbase, _CEG)], sem).wait()

    def blk_body(blk, carry):
        boff = pl.multiple_of(blk * _IBLK, 8)
        pltpu.sync_copy(src_hbm.at[pl.ds(wbase + boff, _IBLK)], isv)
        pltpu.sync_copy(dst_hbm.at[pl.ds(wbase + boff, _IBLK)], idv)
        issue_g(bAs, bAd, 0, sgA)

        def body(j, carry2):
            kA = 2 * j
            kB = 2 * j + 1
            wait_g(bAs, bAd, sgA)

            @pl.when(j > 0)
            def _():
                wait_w(bBs, bBd, swB)

            issue_g(bBs, bBd, kB, sgB)
            issue_w(bAs, bAd, boff, kA, swA)
            wait_g(bBs, bBd, sgB)
            wait_w(bAs, bAd, swA)

            @pl.when(j < _NPRB - 1)
            def _():
                issue_g(bAs, bAd, kA + 2, sgA)

            issue_w(bBs, bBd, boff, kB, swB)
            return carry2

        lax.fori_loop(0, _NPRB, body, 0)
        wait_w(bBs, bBd, swB)
        return carry

    lax.fori_loop(0, _NBLK, blk_body, 0)


@functools.partial(
    pl.kernel,
    out_type=jax.ShapeDtypeStruct((2 * _NB, _NP, _F), _f32),
    mesh=_SC_MESH,
    scratch_types=[
        pltpu.VMEM_SHARED((_NP, _F), _f32),
        pltpu.VMEM((_CES, _F), _f32),
        pltpu.VMEM((_CES, _F), _f32),
        pltpu.VMEM((_CES,), jnp.int32),
        pltpu.VMEM((_CES,), jnp.int32),
        pltpu.SemaphoreType.DMA,
        pltpu.SemaphoreType.DMA,
        pltpu.SemaphoreType.DMA,
        pltpu.SemaphoreType.DMA,
    ],
)
def _sc_scatter(pay_hbm, dst_hbm, zero_hbm, acc_hbm,
                shacc, pbA, pbB, ixA, ixB, slA, slB, ssA, ssB):
    cid = lax.axis_index("c")
    sid = lax.axis_index("s")
    nbase = sid * _NPS

    def wait_l(pb, ix, sem):
        pltpu.make_async_copy(pay_hbm.at[pl.ds(0, _CES)], pb, sem).wait()
        pltpu.make_async_copy(dst_hbm.at[pl.ds(0, _CES)], ix, sem).wait()

    def wait_s(pb, sem):
        pltpu.make_async_copy(pb, shacc.at[pl.ds(0, _CES)], sem).wait()

    for b in range(_NB):
        ebase = pl.multiple_of((cid * 16 + sid) * _EPS, 8)

        def issue_l(pb, ix, c, sem, _eb=ebase, _b=b):
            off = pl.multiple_of(_eb + c * _CES, 8)
            pltpu.async_copy(pay_hbm.at[pl.ds(_b * _E + off, _CES)], pb, sem)
            pltpu.async_copy(dst_hbm.at[pl.ds(off, _CES)], ix, sem)

        def issue_s(pb, ix, sem):
            pltpu.async_copy(pb, shacc.at[ix], sem, add=True)

        pltpu.sync_copy(zero_hbm.at[pl.ds(nbase, _NPS)],
                        shacc.at[pl.ds(nbase, _NPS)])
        plsc.subcore_barrier()

        issue_l(pbA, ixA, 0, slA)

        def body(j, carry):
            cB = 2 * j + 1
            wait_l(pbA, ixA, slA)

            @pl.when(j > 0)
            def _():
                wait_s(pbB, ssB)

            issue_l(pbB, ixB, cB, slB)
            issue_s(pbA, ixA, ssA)
            wait_l(pbB, ixB, slB)
            wait_s(pbA, ssA)

            @pl.when(j < _NPRS - 1)
            def _():
                issue_l(pbA, ixA, 2 * j + 2, slA)

            issue_s(pbB, ixB, ssB)
            return carry

        lax.fori_loop(0, _NPRS, body, 0)
        wait_s(pbB, ssB)
        issue_l(pbA, ixA, _NCHS - 1, slA)
        wait_l(pbA, ixA, slA)
        issue_s(pbA, ixA, ssA)
        wait_s(pbA, ssA)
        plsc.subcore_barrier()
        pltpu.sync_copy(shacc.at[pl.ds(nbase, _NPS)],
                        acc_hbm.at[2 * b + cid, pl.ds(nbase, _NPS)])
        plsc.subcore_barrier()


def _edge_body(gs_ref, gd_ref, e_ref,
               c_ref, r_ref, b1_ref, w2_ref, b2_ref, wx_ref, bx_ref,
               pay_ref, en_ref):
    r_v = r_ref[0]
    b1_v = b1_ref[0]
    b2_v = b2_ref[0]
    bx_v = bx_ref[0]
    gs = gs_ref[...]
    gd = gd_ref[...]
    himask = _u32(0xFFFF0000)
    sh = _u32(16)
    ps_w = gs[:, 0:64]
    pd_w = gd[:, 0:64]
    lo = (lax.bitcast_convert_type(ps_w << sh, _f32)
          + lax.bitcast_convert_type(pd_w << sh, _f32))
    hi = (lax.bitcast_convert_type(ps_w & himask, _f32)
          + lax.bitcast_convert_type(pd_w & himask, _f32))
    z = jnp.concatenate([lo, hi], axis=1)
    xs = lax.bitcast_convert_type(gs[:, 64:67], _f32)
    xd = lax.bitcast_convert_type(gd[:, 64:67], _f32)
    rel = xs - xd
    d2 = jnp.sum(rel * rel, axis=1, keepdims=True)
    z = z + jnp.dot(e_ref[...], c_ref[...], preferred_element_type=_f32)
    z = z + d2 * r_v + b1_v
    m = _silu(z)
    en = jnp.dot(m, w2_ref[...], preferred_element_type=_f32) + b2_v
    en = _silu(en)
    w = jnp.tanh(jnp.dot(en, wx_ref[...], preferred_element_type=_f32)
                 + bx_v)[:, 0:1]
    relw = rel * w
    ones = jnp.ones((_TE, 1), _f32)
    zeros = jnp.zeros((_TE, 108), _f32)
    pay_ref[...] = jnp.concatenate([en, relw, ones, zeros], axis=1)
    en_ref[...] = en


def _tc_edge(Gs, Gd, e, C, r, b1, W2, b2, Wx8, bx8):
    nt = _E // _TE
    bi = lambda b, i: (b * nt + i, 0)
    wb = lambda b, i: (b, 0)
    wb3 = lambda b, i: (b, 0, 0)
    return pl.pallas_call(
        _edge_body,
        grid=(_NB, nt),
        in_specs=[
            pl.BlockSpec((_TE, _F), bi),
            pl.BlockSpec((_TE, _F), bi),
            pl.BlockSpec((_TE, 16), bi),
            pl.BlockSpec((_De, _H), wb),
            pl.BlockSpec((1, 1, _H), wb3),
            pl.BlockSpec((1, 1, _H), wb3),
            pl.BlockSpec((_H, _De), wb),
            pl.BlockSpec((1, 1, _De), wb3),
            pl.BlockSpec((_De, 8), wb),
            pl.BlockSpec((1, 1, 8), wb3),
        ],
        out_specs=[
            pl.BlockSpec((_TE, _F), bi),
            pl.BlockSpec((_TE, 16), bi),
        ],
        out_shape=[
            jax.ShapeDtypeStruct((_NE4, _F), _f32),
            jax.ShapeDtypeStruct((_NE4, 16), _f32),
        ],
    )(Gs, Gd, e, C, r, b1, W2, b2, Wx8, bx8)


def _node_body(h_ref, xp_ref, acc_ref, w1h_ref, w1a_ref, b1_ref, w2_ref,
               b2_ref, hn_ref, xn_ref):
    b1_v = b1_ref[0]
    b2_v = b2_ref[0]
    acc = acc_ref[0] + acc_ref[1]
    deg = jnp.maximum(acc[:, 19:20], 1.0)
    agg = acc[:, 0:16] / deg
    dx = acc[:, 16:19] / deg
    xn_ref[...] = xp_ref[...] + jnp.concatenate(
        [dx, jnp.zeros((_TN, 125), _f32)], axis=1)
    h = h_ref[...]
    pre = (jnp.dot(h, w1h_ref[...], preferred_element_type=_f32)
           + jnp.dot(agg, w1a_ref[...], preferred_element_type=_f32)
           + b1_v)
    hn_ref[...] = h + jnp.dot(_silu(pre), w2_ref[...],
                              preferred_element_type=_f32) + b2_v


def _tc_node(h, xp, acc2, W1h, W1a, b1, W2, b2):
    nt = _N // _TN
    bi = lambda b, i: (b * nt + i, 0)
    wb = lambda b, i: (b, 0)
    wb3 = lambda b, i: (b, 0, 0)
    return pl.pallas_call(
        _node_body,
        grid=(_NB, nt),
        in_specs=[
            pl.BlockSpec((_TN, _F), bi),
            pl.BlockSpec((_TN, _F), bi),
            pl.BlockSpec((2, _TN, _F), lambda b, i: (b, i, 0)),
            pl.BlockSpec((_F, _H), wb),
            pl.BlockSpec((16, _H), wb),
            pl.BlockSpec((1, 1, _H), wb3),
            pl.BlockSpec((_H, _F), wb),
            pl.BlockSpec((1, 1, _F), wb3),
        ],
        out_specs=[
            pl.BlockSpec((_TN, _F), bi),
            pl.BlockSpec((_TN, _F), bi),
        ],
        out_shape=[
            jax.ShapeDtypeStruct((_N4, _F), _f32),
            jax.ShapeDtypeStruct((_N4, _F), _f32),
        ],
    )(h, xp, acc2, W1h, W1a, b1, W2, b2)


def _head_body(hm_ref, hd_ref, hq_ref, ho_ref, nf_ref, gid_ref,
               wm_ref, bm_ref, wd_ref, bd_ref, wq_ref, bq_ref, wo_ref, bo_ref,
               pm_ref, pd_ref, pq_ref, po_ref, gs_ref):
    pm = jnp.dot(hm_ref[...], wm_ref[...], preferred_element_type=_f32) + bm_ref[...]
    mask = nf_ref[:, 0:1] == 1.0
    pm = jnp.where(mask, jnp.abs(pm), pm)
    pm_ref[...] = pm

    pd_ref[...] = jnp.dot(hd_ref[...], wd_ref[...],
                          preferred_element_type=_f32) + bd_ref[...]

    pq = jnp.dot(hq_ref[...], wq_ref[...], preferred_element_type=_f32) + bq_ref[...]
    i8 = lax.broadcasted_iota(jnp.int32, (1, 8), 1)
    mq = ((i8 == 0) | (i8 == 3) | (i8 == 5)).astype(_f32)
    mt = (pq[:, 0:1] + pq[:, 3:4] + pq[:, 5:6]) / 3.0
    pq_ref[...] = pq - mt * mq

    po = jnp.dot(ho_ref[...], wo_ref[...], preferred_element_type=_f32) + bo_ref[...]
    i16 = lax.broadcasted_iota(jnp.int32, (1, 16), 1)
    m0 = ((i16 == 0) | (i16 == 3) | (i16 == 5)).astype(_f32)
    m1 = ((i16 == 6) | (i16 == 1) | (i16 == 8)).astype(_f32)
    m2 = ((i16 == 9) | (i16 == 2) | (i16 == 7)).astype(_f32)
    t0 = (po[:, 0:1] + po[:, 3:4] + po[:, 5:6]) / 3.0
    t1 = (po[:, 6:7] + po[:, 1:2] + po[:, 8:9]) / 3.0
    t2 = (po[:, 9:10] + po[:, 2:3] + po[:, 7:8]) / 3.0
    po_ref[...] = po - t0 * m0 - t1 * m1 - t2 * m2

    gid = gid_ref[...]
    onehot = (lax.broadcasted_iota(jnp.int32, (_TN, _GP), 1) == gid).astype(_f32)
    ssum = lax.dot_general(onehot, pm[:, 0:1], (((0,), (0,)), ((), ())),
                           preferred_element_type=_f32)
    cnt = lax.dot_general(onehot, jnp.ones((_TN, 1), _f32),
                          (((0,), (0,)), ((), ())), preferred_element_type=_f32)
    locg = jnp.concatenate([ssum, cnt, jnp.zeros((_GP, 6), _f32)], axis=1)

    @pl.when(pl.program_id(0) == 0)
    def _():
        gs_ref[...] = locg

    @pl.when(pl.program_id(0) > 0)
    def _():
        gs_ref[...] = gs_ref[...] + locg


def _tc_head(hm, hd, hq, ho, nf, gid2, Wm8, bm8, Wd8, bd8, Wq8, bq8, Wo16, bo16):
    zz = lambda i: (0, 0)
    ii = lambda i: (i, 0)
    return pl.pallas_call(
        _head_body,
        grid=(_N // _TN,),
        in_specs=[
            pl.BlockSpec((_TN, _F), ii),
            pl.BlockSpec((_TN, _F), ii),
            pl.BlockSpec((_TN, _F), ii),
            pl.BlockSpec((_TN, _F), ii),
            pl.BlockSpec((_TN, _F), ii),
            pl.BlockSpec((_TN, 1), ii),
            pl.BlockSpec((_F, 8), zz),
            pl.BlockSpec((1, 8), zz),
            pl.BlockSpec((_F, 8), zz),
            pl.BlockSpec((1, 8), zz),
            pl.BlockSpec((_F, 8), zz),
            pl.BlockSpec((1, 8), zz),
            pl.BlockSpec((_F, 16), zz),
            pl.BlockSpec((1, 16), zz),
        ],
        out_specs=[
            pl.BlockSpec((_TN, 8), ii),
            pl.BlockSpec((_TN, 8), ii),
            pl.BlockSpec((_TN, 8), ii),
            pl.BlockSpec((_TN, 16), ii),
            pl.BlockSpec((_GP, 8), zz),
        ],
        out_shape=[
            jax.ShapeDtypeStruct((_N, 8), _f32),
            jax.ShapeDtypeStruct((_N, 8), _f32),
            jax.ShapeDtypeStruct((_N, 8), _f32),
            jax.ShapeDtypeStruct((_N, 16), _f32),
            jax.ShapeDtypeStruct((_GP, 8), _f32),
        ],
    )(hm, hd, hq, ho, nf, gid2, Wm8, bm8, Wd8, bd8, Wq8, bq8, Wo16, bo16)


def _fix_body(pm_ref, gid_ref, gs_ref, out_ref):
    sums = gs_ref[:, 0:1]
    cnts = jnp.maximum(gs_ref[:, 1:2], 1.0)
    fv = jnp.where(jnp.abs(sums) < 0.01, 0.0, sums / cnts)
    gid = gid_ref[...]
    onehot = (lax.broadcasted_iota(jnp.int32, (_TN, _GP), 1) == gid).astype(_f32)
    fvg = jnp.dot(onehot, fv, preferred_element_type=_f32)
    out_ref[...] = pm_ref[...] - fvg


def _tc_fix(pm_raw, gid2, gsums):
    ii = lambda i: (i, 0)
    return pl.pallas_call(
        _fix_body,
        grid=(_N // _TN,),
        in_specs=[
            pl.BlockSpec((_TN, 8), ii),
            pl.BlockSpec((_TN, 1), ii),
            pl.BlockSpec((_GP, 8), lambda i: (0, 0)),
        ],
        out_specs=pl.BlockSpec((_TN, 8), ii),
        out_shape=jax.ShapeDtypeStruct((_N, 8), _f32),
    )(pm_raw, gid2, gsums)


def kernel(nfeats, coordinates, efeats, edge_index, node_graph_ids,
           We1, be1, We2, be2, Wx, bx, Wh1, bh1, Wh2, bh2,
           Wm, bm, Wd, bd, Wq, bq, Wo, bo):
    src = edge_index[0]
    dst = edge_index[1]
    boff = (jnp.arange(_NB, dtype=jnp.int32) * _N)[:, None]
    src4 = (src[None, :] + boff).reshape(-1)
    dst4 = (dst[None, :] + boff).reshape(-1)

    sigma = np.concatenate([np.arange(0, _H, 2), np.arange(1, _H, 2)])

    A_all = We1[:, :_F, :]
    B_all = We1[:, _F:2 * _F, :]
    C_all = We1[:, 2 * _F:2 * _F + _De, :][:, :, sigma]
    r_all = We1[:, 2 * _F + _De, :][:, sigma]
    b1_all = be1[:, sigma]
    W2_all = We2[:, sigma, :]
    Wx8 = jnp.concatenate([Wx, jnp.zeros((_NCONV, _De, 7), _f32)], axis=2)
    bx8 = jnp.concatenate([bx.reshape(_NCONV, 1, 1),
                           jnp.zeros((_NCONV, 1, 7), _f32)], axis=2)
    W1h_all = Wh1[:, :_F, :]
    W1a_all = Wh1[:, _F:, :]

    xp0 = jnp.concatenate([coordinates, jnp.zeros((_N, 125), _f32)], axis=1)
    zeros_acc = jnp.zeros((_NP, _F), _f32)

    h_st = jnp.tile(nfeats, (_NB, 1))
    xp_st = jnp.tile(xp0, (_NB, 1))
    e_st = jnp.tile(efeats, (_NB, 1))

    for l in range(5):
        ids = np.array([l, 5 + l, 10 + l, 15 + l])
        A_s = A_all[ids].reshape(_NB * _F, _F)
        B_s = B_all[ids].reshape(_NB * _F, _F)
        C_s = C_all[ids].reshape(_NB * _De, _H)
        r_s = r_all[ids].reshape(_NB, 1, _H)
        b1_s = b1_all[ids].reshape(_NB, 1, _H)
        W2_s = W2_all[ids].reshape(_NB * _H, _De)
        b2_s = be2[ids].reshape(_NB, 1, _De)
        Wx_s = Wx8[ids].reshape(_NB * _De, 8)
        bx_s = bx8[ids].reshape(_NB, 1, 8)
        W1h_s = W1h_all[ids].reshape(_NB * _F, _H)
        W1a_s = W1a_all[ids].reshape(_NB * _De, _H)
        bh1_s = bh1[ids].reshape(_NB, 1, _H)
        Wh2_s = Wh2[ids].reshape(_NB * _H, _F)
        bh2_s = bh2[ids].reshape(_NB, 1, _F)

        Ps, Pd = _tc_prep(h_st, A_s, B_s)
        Ts = _pack_table(Ps, xp_st)
        Td = _pack_table(Pd, xp_st)
        Gs, Gd = _sc_gather(Ts, Td, src4, dst4)
        pay, e_st = _tc_edge(Gs, Gd, e_st, C_s, r_s, b1_s,
                             W2_s, b2_s, Wx_s, bx_s)
        acc = _sc_scatter(pay, dst, zeros_acc)
        h_st, xp_st = _tc_node(h_st, xp_st, acc, W1h_s, W1a_s, bh1_s,
                               Wh2_s, bh2_s)

    h_mon = h_st[0:_N]
    h_dip = h_st[_N:2 * _N]
    h_quad = h_st[2 * _N:3 * _N]
    h_oct = h_st[3 * _N:4 * _N]

    gid2 = node_graph_ids.reshape(_N, 1).astype(jnp.int32)
    Wm8 = jnp.concatenate([Wm, jnp.zeros((_F, 7), _f32)], axis=1)
    bm8 = jnp.concatenate([bm.reshape(1, 1), jnp.zeros((1, 7), _f32)], axis=1)
    Wd8 = jnp.concatenate([Wd, jnp.zeros((_F, 5), _f32)], axis=1)
    bd8 = jnp.concatenate([bd.reshape(1, 3), jnp.zeros((1, 5), _f32)], axis=1)
    Wq8 = jnp.concatenate([Wq, jnp.zeros((_F, 2), _f32)], axis=1)
    bq8 = jnp.concatenate([bq.reshape(1, 6), jnp.zeros((1, 2), _f32)], axis=1)
    Wo16 = jnp.concatenate([Wo, jnp.zeros((_F, 6), _f32)], axis=1)
    bo16 = jnp.concatenate([bo.reshape(1, 10), jnp.zeros((1, 6), _f32)], axis=1)

    pm_raw, pd, pq, po, gsums = _tc_head(
        h_mon, h_dip, h_quad, h_oct, nfeats, gid2,
        Wm8, bm8, Wd8, bd8, Wq8, bq8, Wo16, bo16)
    pm = _tc_fix(pm_raw, gid2, gsums)

    return jnp.concatenate(
        [pm[:, :1], pd[:, :3], pq[:, :6], po[:, :10]], axis=1)


# speedup vs baseline: 1.2109x; 1.2109x over previous
import functools

import jax
import jax.numpy as jnp
import numpy as np
from jax import lax
from jax.experimental import pallas as pl
from jax.experimental.pallas import tpu as pltpu
import jax.experimental.pallas.tpu_sc as plsc

_N = 10000
_E = 320000
_F = 128
_De = 16
_H = 128
_G = 100
_NCONV = 20

_TN = 2000
_TE = 2000
_GP = 104

_NW = 32
_EPW = _E // _NW
_CEG = 200
_NCHG = _EPW // _CEG
_NPRG = _NCHG // 2
_CES = 80
_NCHS = _EPW // _CES
_NPRS = (_NCHS - 1) // 2
_NP = 10240
_NPS = _NP // 16

_f32 = jnp.float32
_u32 = jnp.uint32
_bf16 = jnp.bfloat16


def _silu(x):
    return x * jax.nn.sigmoid(x)


def _prep_body(h_ref, a_ref, b_ref, ps_ref, pd_ref):
    h = h_ref[...]
    ps_ref[...] = jnp.dot(h, a_ref[...], preferred_element_type=_f32)
    pd_ref[...] = jnp.dot(h, b_ref[...], preferred_element_type=_f32)


def _tc_prep(h, A, B):
    return pl.pallas_call(
        _prep_body,
        grid=(_N // _TN,),
        in_specs=[
            pl.BlockSpec((_TN, _F), lambda i: (i, 0)),
            pl.BlockSpec((_F, _F), lambda i: (0, 0)),
            pl.BlockSpec((_F, _F), lambda i: (0, 0)),
        ],
        out_specs=[
            pl.BlockSpec((_TN, _F), lambda i: (i, 0)),
            pl.BlockSpec((_TN, _F), lambda i: (i, 0)),
        ],
        out_shape=[jax.ShapeDtypeStruct((_N, _F), _f32)] * 2,
    )(h, A, B)


def _pack_table(P, xp):
    pw = lax.bitcast_convert_type(
        P.astype(_bf16).reshape(_N, _F // 2, 2), _u32)
    xw = lax.bitcast_convert_type(xp[:, 0:3], _u32)
    return jnp.concatenate(
        [pw, xw, jnp.zeros((_N, 61), _u32)], axis=1)


_SC_MESH = plsc.VectorSubcoreMesh(core_axis_name="c", subcore_axis_name="s")


@functools.partial(
    pl.kernel,
    out_type=[
        jax.ShapeDtypeStruct((_E, _F), _u32),
        jax.ShapeDtypeStruct((_E, _F), _u32),
    ],
    mesh=_SC_MESH,
    scratch_types=[
        pltpu.VMEM((_EPW,), jnp.int32),
        pltpu.VMEM((_EPW,), jnp.int32),
        pltpu.VMEM((_CEG, _F), _u32),
        pltpu.VMEM((_CEG, _F), _u32),
        pltpu.VMEM((_CEG, _F), _u32),
        pltpu.VMEM((_CEG, _F), _u32),
        pltpu.SemaphoreType.DMA,
        pltpu.SemaphoreType.DMA,
        pltpu.SemaphoreType.DMA,
        pltpu.SemaphoreType.DMA,
    ],
)
def _sc_gather(ts_hbm, td_hbm, src_hbm, dst_hbm,
               gs_hbm, gd_hbm,
               isv, idv, bAs, bAd, bBs, bBd, sgA, sgB, swA, swB):
    cid = lax.axis_index("c")
    sid = lax.axis_index("s")
    wbase = pl.multiple_of((cid * 16 + sid) * _EPW, 8)
    pltpu.sync_copy(src_hbm.at[pl.ds(wbase, _EPW)], isv)
    pltpu.sync_copy(dst_hbm.at[pl.ds(wbase, _EPW)], idv)

    def issue_g(bs, bd, c, sem):
        off = pl.multiple_of(c * _CEG, 8)
        pltpu.async_copy(ts_hbm.at[isv.at[pl.ds(off, _CEG)]], bs, sem)
        pltpu.async_copy(td_hbm.at[idv.at[pl.ds(off, _CEG)]], bd, sem)

    def wait_g(bs, bd, sem):
        pltpu.make_async_copy(ts_hbm.at[pl.ds(0, _CEG)], bs, sem).wait()
        pltpu.make_async_copy(td_hbm.at[pl.ds(0, _CEG)], bd, sem).wait()

    def issue_w(bs, bd, c, sem):
        base = pl.multiple_of(wbase + c * _CEG, 8)
        pltpu.async_copy(bs, gs_hbm.at[pl.ds(base, _CEG)], sem)
        pltpu.async_copy(bd, gd_hbm.at[pl.ds(base, _CEG)], sem)

    def wait_w(bs, bd, sem):
        pltpu.make_async_copy(bs, gs_hbm.at[pl.ds(wbase, _CEG)], sem).wait()
        pltpu.make_async_copy(bd, gd_hbm.at[pl.ds(wbase, _CEG)], sem).wait()

    issue_g(bAs, bAd, 0, sgA)

    def body(j, carry):
        cA = 2 * j
        cB = 2 * j + 1
        wait_g(bAs, bAd, sgA)

        @pl.when(j > 0)
        def _():
            wait_w(bBs, bBd, swB)

        issue_g(bBs, bBd, cB, sgB)
        issue_w(bAs, bAd, cA, swA)
        wait_g(bBs, bBd, sgB)
        wait_w(bAs, bAd, swA)

        @pl.when(j < _NPRG - 1)
        def _():
            issue_g(bAs, bAd, cA + 2, sgA)

        issue_w(bBs, bBd, cB, swB)
        return carry

    lax.fori_loop(0, _NPRG, body, 0)
    wait_w(bBs, bBd, swB)


@functools.partial(
    pl.kernel,
    out_type=jax.ShapeDtypeStruct((2, _NP, _F), _f32),
    mesh=_SC_MESH,
    scratch_types=[
        pltpu.VMEM_SHARED((_NP, _F), _f32),
        pltpu.VMEM((_CES, _F), _f32),
        pltpu.VMEM((_CES, _F), _f32),
        pltpu.VMEM((_CES,), jnp.int32),
        pltpu.VMEM((_CES,), jnp.int32),
        pltpu.SemaphoreType.DMA,
        pltpu.SemaphoreType.DMA,
        pltpu.SemaphoreType.DMA,
        pltpu.SemaphoreType.DMA,
    ],
)
def _sc_scatter(pay_hbm, dst_hbm, zero_hbm, acc_hbm,
                shacc, pbA, pbB, ixA, ixB, slA, slB, ssA, ssB):
    cid = lax.axis_index("c")
    sid = lax.axis_index("s")
    nbase = sid * _NPS
    wbase = pl.multiple_of((cid * 16 + sid) * _EPW, 8)
    pltpu.sync_copy(zero_hbm.at[pl.ds(nbase, _NPS)], shacc.at[pl.ds(nbase, _NPS)])
    plsc.subcore_barrier()

    def issue_l(pb, ix, c, sem):
        base = pl.multiple_of(wbase + c * _CES, 8)
        pltpu.async_copy(pay_hbm.at[pl.ds(base, _CES)], pb, sem)
        pltpu.async_copy(dst_hbm.at[pl.ds(base, _CES)], ix, sem)

    def wait_l(pb, ix, sem):
        pltpu.make_async_copy(pay_hbm.at[pl.ds(wbase, _CES)], pb, sem).wait()
        pltpu.make_async_copy(dst_hbm.at[pl.ds(wbase, _CES)], ix, sem).wait()

    def issue_s(pb, ix, sem):
        pltpu.async_copy(pb, shacc.at[ix], sem, add=True)

    def wait_s(pb, ix, sem):
        pltpu.make_async_copy(pb, shacc.at[pl.ds(0, _CES)], sem).wait()

    issue_l(pbA, ixA, 0, slA)

    def body(j, carry):
        cA = 2 * j
        cB = 2 * j + 1
        wait_l(pbA, ixA, slA)

        @pl.when(j > 0)
        def _():
            wait_s(pbB, ixB, ssB)

        issue_l(pbB, ixB, cB, slB)
        issue_s(pbA, ixA, ssA)
        wait_l(pbB, ixB, slB)
        wait_s(pbA, ixA, ssA)

        @pl.when(j < _NPRS - 1)
        def _():
            issue_l(pbA, ixA, cA + 2, slA)

        issue_s(pbB, ixB, ssB)
        return carry

    lax.fori_loop(0, _NPRS, body, 0)
    wait_s(pbB, ixB, ssB)
    issue_l(pbA, ixA, _NCHS - 1, slA)
    wait_l(pbA, ixA, slA)
    issue_s(pbA, ixA, ssA)
    wait_s(pbA, ixA, ssA)
    plsc.subcore_barrier()
    pltpu.sync_copy(shacc.at[pl.ds(nbase, _NPS)], acc_hbm.at[cid, pl.ds(nbase, _NPS)])


def _edge_body(gs_ref, gd_ref, e_ref,
               c_ref, r_ref, b1_ref, w2_ref, b2_ref, wx_ref, bx_ref,
               pay_ref, en_ref):
    gs = gs_ref[...]
    gd = gd_ref[...]
    himask = _u32(0xFFFF0000)
    sh = _u32(16)
    ps_w = gs[:, 0:64]
    pd_w = gd[:, 0:64]
    lo = (lax.bitcast_convert_type(ps_w << sh, _f32)
          + lax.bitcast_convert_type(pd_w << sh, _f32))
    hi = (lax.bitcast_convert_type(ps_w & himask, _f32)
          + lax.bitcast_convert_type(pd_w & himask, _f32))
    z = jnp.concatenate([lo, hi], axis=1)
    xs = lax.bitcast_convert_type(gs[:, 64:67], _f32)
    xd = lax.bitcast_convert_type(gd[:, 64:67], _f32)
    rel = xs - xd
    d2 = jnp.sum(rel * rel, axis=1, keepdims=True)
    z = z + jnp.dot(e_ref[...], c_ref[...], preferred_element_type=_f32)
    z = z + d2 * r_ref[...] + b1_ref[...]
    m = _silu(z)
    en = jnp.dot(m, w2_ref[...], preferred_element_type=_f32) + b2_ref[...]
    en = _silu(en)
    w = jnp.tanh(jnp.dot(en, wx_ref[...], preferred_element_type=_f32)
                 + bx_ref[...])[:, 0:1]
    relw = rel * w
    ones = jnp.ones((_TE, 1), _f32)
    zeros = jnp.zeros((_TE, 108), _f32)
    pay_ref[...] = jnp.concatenate([en, relw, ones, zeros], axis=1)
    en_ref[...] = en


def _tc_edge(Gs, Gd, e, C, r, b1, W2, b2, Wx8, bx8):
    zz = lambda i: (0, 0)
    return pl.pallas_call(
        _edge_body,
        grid=(_E // _TE,),
        in_specs=[
            pl.BlockSpec((_TE, _F), lambda i: (i, 0)),
            pl.BlockSpec((_TE, _F), lambda i: (i, 0)),
            pl.BlockSpec((_TE, 16), lambda i: (i, 0)),
            pl.BlockSpec((_De, _H), zz),
            pl.BlockSpec((1, _H), zz),
            pl.BlockSpec((1, _H), zz),
            pl.BlockSpec((_H, _De), zz),
            pl.BlockSpec((1, _De), zz),
            pl.BlockSpec((_De, 8), zz),
            pl.BlockSpec((1, 8), zz),
        ],
        out_specs=[
            pl.BlockSpec((_TE, _F), lambda i: (i, 0)),
            pl.BlockSpec((_TE, 16), lambda i: (i, 0)),
        ],
        out_shape=[
            jax.ShapeDtypeStruct((_E, _F), _f32),
            jax.ShapeDtypeStruct((_E, 16), _f32),
        ],
    )(Gs, Gd, e, C, r, b1, W2, b2, Wx8, bx8)


def _node_body(h_ref, xp_ref, acc_ref, w1h_ref, w1a_ref, b1_ref, w2_ref,
               b2_ref, hn_ref, xn_ref):
    acc = acc_ref[0] + acc_ref[1]
    deg = jnp.maximum(acc[:, 19:20], 1.0)
    agg = acc[:, 0:16] / deg
    dx = acc[:, 16:19] / deg
    xn_ref[...] = xp_ref[...] + jnp.concatenate(
        [dx, jnp.zeros((_TN, 125), _f32)], axis=1)
    h = h_ref[...]
    pre = (jnp.dot(h, w1h_ref[...], preferred_element_type=_f32)
           + jnp.dot(agg, w1a_ref[...], preferred_element_type=_f32)
           + b1_ref[...])
    hn_ref[...] = h + jnp.dot(_silu(pre), w2_ref[...],
                              preferred_element_type=_f32) + b2_ref[...]


def _tc_node(h, xp, acc2, W1h, W1a, b1, W2, b2):
    zz = lambda i: (0, 0)
    return pl.pallas_call(
        _node_body,
        grid=(_N // _TN,),
        in_specs=[
            pl.BlockSpec((_TN, _F), lambda i: (i, 0)),
            pl.BlockSpec((_TN, _F), lambda i: (i, 0)),
            pl.BlockSpec((2, _TN, _F), lambda i: (0, i, 0)),
            pl.BlockSpec((_F, _H), zz),
            pl.BlockSpec((16, _H), zz),
            pl.BlockSpec((1, _H), zz),
            pl.BlockSpec((_H, _F), zz),
            pl.BlockSpec((1, _F), zz),
        ],
        out_specs=[
            pl.BlockSpec((_TN, _F), lambda i: (i, 0)),
            pl.BlockSpec((_TN, _F), lambda i: (i, 0)),
        ],
        out_shape=[
            jax.ShapeDtypeStruct((_N, _F), _f32),
            jax.ShapeDtypeStruct((_N, _F), _f32),
        ],
    )(h, xp, acc2, W1h, W1a, b1, W2, b2)


def _head_body(hm_ref, hd_ref, hq_ref, ho_ref, nf_ref, gid_ref,
               wm_ref, bm_ref, wd_ref, bd_ref, wq_ref, bq_ref, wo_ref, bo_ref,
               pm_ref, pd_ref, pq_ref, po_ref, gs_ref):
    pm = jnp.dot(hm_ref[...], wm_ref[...], preferred_element_type=_f32) + bm_ref[...]
    mask = nf_ref[:, 0:1] == 1.0
    pm = jnp.where(mask, jnp.abs(pm), pm)
    pm_ref[...] = pm

    pd_ref[...] = jnp.dot(hd_ref[...], wd_ref[...],
                          preferred_element_type=_f32) + bd_ref[...]

    pq = jnp.dot(hq_ref[...], wq_ref[...], preferred_element_type=_f32) + bq_ref[...]
    i8 = lax.broadcasted_iota(jnp.int32, (1, 8), 1)
    mq = ((i8 == 0) | (i8 == 3) | (i8 == 5)).astype(_f32)
    mt = (pq[:, 0:1] + pq[:, 3:4] + pq[:, 5:6]) / 3.0
    pq_ref[...] = pq - mt * mq

    po = jnp.dot(ho_ref[...], wo_ref[...], preferred_element_type=_f32) + bo_ref[...]
    i16 = lax.broadcasted_iota(jnp.int32, (1, 16), 1)
    m0 = ((i16 == 0) | (i16 == 3) | (i16 == 5)).astype(_f32)
    m1 = ((i16 == 6) | (i16 == 1) | (i16 == 8)).astype(_f32)
    m2 = ((i16 == 9) | (i16 == 2) | (i16 == 7)).astype(_f32)
    t0 = (po[:, 0:1] + po[:, 3:4] + po[:, 5:6]) / 3.0
    t1 = (po[:, 6:7] + po[:, 1:2] + po[:, 8:9]) / 3.0
    t2 = (po[:, 9:10] + po[:, 2:3] + po[:, 7:8]) / 3.0
    po_ref[...] = po - t0 * m0 - t1 * m1 - t2 * m2

    gid = gid_ref[...]
    onehot = (lax.broadcasted_iota(jnp.int32, (_TN, _GP), 1) == gid).astype(_f32)
    ssum = lax.dot_general(onehot, pm[:, 0:1], (((0,), (0,)), ((), ())),
                           preferred_element_type=_f32)
    cnt = lax.dot_general(onehot, jnp.ones((_TN, 1), _f32),
                          (((0,), (0,)), ((), ())), preferred_element_type=_f32)
    locg = jnp.concatenate([ssum, cnt, jnp.zeros((_GP, 6), _f32)], axis=1)

    @pl.when(pl.program_id(0) == 0)
    def _():
        gs_ref[...] = locg

    @pl.when(pl.program_id(0) > 0)
    def _():
        gs_ref[...] = gs_ref[...] + locg


def _tc_head(hm, hd, hq, ho, nf, gid2, Wm8, bm8, Wd8, bd8, Wq8, bq8, Wo16, bo16):
    zz = lambda i: (0, 0)
    ii = lambda i: (i, 0)
    return pl.pallas_call(
        _head_body,
        grid=(_N // _TN,),
        in_specs=[
            pl.BlockSpec((_TN, _F), ii),
            pl.BlockSpec((_TN, _F), ii),
            pl.BlockSpec((_TN, _F), ii),
            pl.BlockSpec((_TN, _F), ii),
            pl.BlockSpec((_TN, _F), ii),
            pl.BlockSpec((_TN, 1), ii),
            pl.BlockSpec((_F, 8), zz),
            pl.BlockSpec((1, 8), zz),
            pl.BlockSpec((_F, 8), zz),
            pl.BlockSpec((1, 8), zz),
            pl.BlockSpec((_F, 8), zz),
            pl.BlockSpec((1, 8), zz),
            pl.BlockSpec((_F, 16), zz),
            pl.BlockSpec((1, 16), zz),
        ],
        out_specs=[
            pl.BlockSpec((_TN, 8), ii),
            pl.BlockSpec((_TN, 8), ii),
            pl.BlockSpec((_TN, 8), ii),
            pl.BlockSpec((_TN, 16), ii),
            pl.BlockSpec((_GP, 8), zz),
        ],
        out_shape=[
            jax.ShapeDtypeStruct((_N, 8), _f32),
            jax.ShapeDtypeStruct((_N, 8), _f32),
            jax.ShapeDtypeStruct((_N, 8), _f32),
            jax.ShapeDtypeStruct((_N, 16), _f32),
            jax.ShapeDtypeStruct((_GP, 8), _f32),
        ],
    )(hm, hd, hq, ho, nf, gid2, Wm8, bm8, Wd8, bd8, Wq8, bq8, Wo16, bo16)


def _fix_body(pm_ref, gid_ref, gs_ref, out_ref):
    sums = gs_ref[:, 0:1]
    cnts = jnp.maximum(gs_ref[:, 1:2], 1.0)
    fv = jnp.where(jnp.abs(sums) < 0.01, 0.0, sums / cnts)
    gid = gid_ref[...]
    onehot = (lax.broadcasted_iota(jnp.int32, (_TN, _GP), 1) == gid).astype(_f32)
    fvg = jnp.dot(onehot, fv, preferred_element_type=_f32)
    out_ref[...] = pm_ref[...] - fvg


def _tc_fix(pm_raw, gid2, gsums):
    ii = lambda i: (i, 0)
    return pl.pallas_call(
        _fix_body,
        grid=(_N // _TN,),
        in_specs=[
            pl.BlockSpec((_TN, 8), ii),
            pl.BlockSpec((_TN, 1), ii),
            pl.BlockSpec((_GP, 8), lambda i: (0, 0)),
        ],
        out_specs=pl.BlockSpec((_TN, 8), ii),
        out_shape=jax.ShapeDtypeStruct((_N, 8), _f32),
    )(pm_raw, gid2, gsums)


def kernel(nfeats, coordinates, efeats, edge_index, node_graph_ids,
           We1, be1, We2, be2, Wx, bx, Wh1, bh1, Wh2, bh2,
           Wm, bm, Wd, bd, Wq, bq, Wo, bo):
    src = edge_index[0]
    dst = edge_index[1]

    sigma = np.concatenate([np.arange(0, _H, 2), np.arange(1, _H, 2)])

    A_all = We1[:, :_F, :]
    B_all = We1[:, _F:2 * _F, :]
    C_all = We1[:, 2 * _F:2 * _F + _De, :][:, :, sigma]
    r_all = We1[:, 2 * _F + _De, :][:, sigma].reshape(_NCONV, 1, _H)
    b1_all = be1[:, sigma].reshape(_NCONV, 1, _H)
    W2_all = We2[:, sigma, :]
    b2_all = be2.reshape(_NCONV, 1, _De)
    Wx8 = jnp.concatenate([Wx, jnp.zeros((_NCONV, _De, 7), _f32)], axis=2)
    bx8 = jnp.concatenate([bx.reshape(_NCONV, 1, 1),
                           jnp.zeros((_NCONV, 1, 7), _f32)], axis=2)
    W1h_all = Wh1[:, :_F, :]
    W1a_all = Wh1[:, _F:, :]
    bh1r = bh1.reshape(_NCONV, 1, _H)
    bh2r = bh2.reshape(_NCONV, 1, _F)

    xp0 = jnp.concatenate([coordinates, jnp.zeros((_N, 125), _f32)], axis=1)
    zeros_acc = jnp.zeros((_NP, _F), _f32)

    def layer(i, h, xp, e):
        Ps, Pd = _tc_prep(h, A_all[i], B_all[i])
        Ts = _pack_table(Ps, xp)
        Td = _pack_table(Pd, xp)
        Gs, Gd = _sc_gather(Ts, Td, src, dst)
        pay, en = _tc_edge(Gs, Gd, e, C_all[i], r_all[i], b1_all[i],
                           W2_all[i], b2_all[i], Wx8[i], bx8[i])
        acc2 = _sc_scatter(pay, dst, zeros_acc)
        h2, xp2 = _tc_node(h, xp, acc2, W1h_all[i], W1a_all[i], bh1r[i],
                           Wh2[i], bh2r[i])
        return h2, xp2, en

    states = [(nfeats, xp0, efeats) for _ in range(4)]
    for l in range(5):
        for b in range(4):
            h, xp, e = states[b]
            states[b] = layer(b * 5 + l, h, xp, e)

    h_mon = states[0][0]
    h_dip = states[1][0]
    h_quad = states[2][0]
    h_oct = states[3][0]

    gid2 = node_graph_ids.reshape(_N, 1).astype(jnp.int32)
    Wm8 = jnp.concatenate([Wm, jnp.zeros((_F, 7), _f32)], axis=1)
    bm8 = jnp.concatenate([bm.reshape(1, 1), jnp.zeros((1, 7), _f32)], axis=1)
    Wd8 = jnp.concatenate([Wd, jnp.zeros((_F, 5), _f32)], axis=1)
    bd8 = jnp.concatenate([bd.reshape(1, 3), jnp.zeros((1, 5), _f32)], axis=1)
    Wq8 = jnp.concatenate([Wq, jnp.zeros((_F, 2), _f32)], axis=1)
    bq8 = jnp.concatenate([bq.reshape(1, 6), jnp.zeros((1, 2), _f32)], axis=1)
    Wo16 = jnp.concatenate([Wo, jnp.zeros((_F, 6), _f32)], axis=1)
    bo16 = jnp.concatenate([bo.reshape(1, 10), jnp.zeros((1, 6), _f32)], axis=1)

    pm_raw, pd, pq, po, gsums = _tc_head(
        h_mon, h_dip, h_quad, h_oct, nfeats, gid2,
        Wm8, bm8, Wd8, bd8, Wq8, bq8, Wo16, bo16)
    pm = _tc_fix(pm_raw, gid2, gsums)

    return jnp.concatenate(
        [pm[:, :1], pd[:, :3], pq[:, :6], po[:, :10]], axis=1)
